# baseline (speedup 1.0000x reference)
import functools

import jax
import jax.numpy as jnp
from jax.experimental import pallas as pl
from jax.experimental.pallas import tpu as pltpu


def _clip_kernel(x_ref, w_ref, mask_ref, tok_ref, emb_ref, pos_ref, wt_ref,
                 ls_ref, loss_ref, acc_ref, ftacc_ref,
                 *, C, P, nh, nw, T, L, inv_b):
    t = pl.program_id(0)
    k = pl.program_id(1)

    @pl.when((t == 0) & (k == 0))
    def _():
        acc_ref[...] = jnp.zeros_like(acc_ref)

    @pl.when(k == 0)
    def _():
        ftacc_ref[...] = jnp.zeros_like(ftacc_ref)

    x = x_ref[0]
    bl = x.shape[-1]
    s1 = jnp.sum(x.reshape(nh * P, nw, P, bl), axis=1)
    s2 = jnp.sum(s1.reshape(nh, P, P, bl), axis=0)
    pp = s2.reshape(P * P, bl).astype(jnp.bfloat16)
    ftacc_ref[...] += jnp.dot(w_ref[0], pp,
                              preferred_element_type=jnp.float32)

    @pl.when(k == C - 1)
    def _():
        ft = ftacc_ref[...]
        ssum = jnp.sum(ft * ft, axis=0, keepdims=True)
        m = mask_ref[0]
        acc_ref[...] += ft * (jax.lax.rsqrt(ssum) * m)

    @pl.when((t == T - 1) & (k == C - 1))
    def _():
        pooled = acc_ref[...]
        pinv = jax.lax.rsqrt(jnp.sum(pooled * pooled, axis=0, keepdims=True))
        vf = pooled * pinv

        tok = tok_ref[...]
        b, v = tok.shape[0], emb_ref.shape[0]
        viota = jax.lax.broadcasted_iota(jnp.int32, (b, v), 1)
        counts = jnp.zeros((b, v), jnp.float32)
        for l in range(L):
            counts += (viota == tok[:, l][:, None]).astype(jnp.float32)
        xt = jnp.dot((counts * (1.0 / L)).astype(jnp.bfloat16),
                     emb_ref[...].astype(jnp.bfloat16),
                     preferred_element_type=jnp.float32)
        xt += jnp.mean(pos_ref[0:L], axis=0, keepdims=True)
        seq = jnp.dot(xt.astype(jnp.bfloat16),
                      wt_ref[...].astype(jnp.bfloat16),
                      preferred_element_type=jnp.float32)
        tinv = jax.lax.rsqrt(jnp.sum(seq * seq, axis=-1, keepdims=True))
        tn = seq * tinv

        scale = jnp.exp(ls_ref[0, 0])
        sim = scale * jnp.dot(tn, vf,
                              preferred_element_type=jnp.float32)
        r = jax.lax.broadcasted_iota(jnp.int32, (b, b), 0)
        c = jax.lax.broadcasted_iota(jnp.int32, (b, b), 1)
        diag = jnp.sum(jnp.where(r == c, sim, 0.0))
        mr = jnp.max(sim, axis=1, keepdims=True)
        racc = jnp.sum(jnp.log(jnp.sum(jnp.exp(sim - mr), axis=1,
                                       keepdims=True)) + mr)
        mc = jnp.max(sim, axis=0, keepdims=True)
        cacc = jnp.sum(jnp.log(jnp.sum(jnp.exp(sim - mc), axis=0,
                                       keepdims=True)) + mc)
        loss = ((racc - diag) + (cacc - diag)) * (0.5 * inv_b)
        loss_ref[...] = jnp.reshape(loss, (1, 1))


def kernel(tok_emb, pos_emb, w_text, w_patch, logit_scale,
           text_input, video, video_mask):
    B, L = text_input.shape
    _, T, C, H, W = video.shape
    D = w_patch.shape[1]
    V, Kt = tok_emb.shape
    P = int(round((w_patch.shape[0] // C) ** 0.5))
    nh, nw = H // P, W // P
    CHW = C * H * W

    wp_t = ((w_patch.T) * (1.0 / (nh * nw))).astype(jnp.bfloat16)
    wp_c = wp_t.reshape(D, C, P * P).transpose(1, 0, 2)

    xs = video.transpose(1, 2, 3, 4, 0).reshape(T, CHW, B)
    mask_t = video_mask.astype(jnp.float32).T.reshape(T, 1, B)

    loss = pl.pallas_call(
        functools.partial(_clip_kernel, C=C, P=P, nh=nh, nw=nw, T=T, L=L,
                          inv_b=1.0 / B),
        out_shape=jax.ShapeDtypeStruct((1, 1), jnp.float32),
        grid_spec=pltpu.PrefetchScalarGridSpec(
            num_scalar_prefetch=0,
            grid=(T, C),
            in_specs=[pl.BlockSpec((1, H * W, B), lambda t, k: (t, k, 0)),
                      pl.BlockSpec((1, D, P * P), lambda t, k: (k, 0, 0)),
                      pl.BlockSpec((1, 1, B), lambda t, k: (t, 0, 0)),
                      pl.BlockSpec((B, L), lambda t, k: (0, 0)),
                      pl.BlockSpec((V, Kt), lambda t, k: (0, 0)),
                      pl.BlockSpec(pos_emb.shape, lambda t, k: (0, 0)),
                      pl.BlockSpec((Kt, D), lambda t, k: (0, 0)),
                      pl.BlockSpec((1, 1), lambda t, k: (0, 0))],
            out_specs=pl.BlockSpec((1, 1), lambda t, k: (0, 0)),
            scratch_shapes=[pltpu.VMEM((D, B), jnp.float32),
                            pltpu.VMEM((D, B), jnp.float32)]),
        compiler_params=pltpu.CompilerParams(
            dimension_semantics=("arbitrary", "arbitrary"),
            vmem_limit_bytes=64 * 1024 * 1024),
        cost_estimate=pl.CostEstimate(
            flops=T * CHW * B + 2 * T * C * P * P * B * D + 2 * B * B * D,
            transcendentals=2 * B * B,
            bytes_accessed=T * CHW * B * 4 + V * Kt * 4),
    )(xs, wp_c, mask_t, text_input, tok_emb, pos_emb, w_text,
      logit_scale.reshape(1, 1))
    return loss[0, 0]


# speedup vs baseline: 1.0732x; 1.0732x over previous
import functools

import jax
import jax.numpy as jnp
from jax.experimental import pallas as pl
from jax.experimental.pallas import tpu as pltpu


def _clip_kernel(x_ref, w_ref, mask_ref, tok_ref, emb_ref, pos_ref, wt_ref,
                 ls_ref, loss_ref, acc_ref, *, C, P, nh, nw, T, L, inv_b):
    t = pl.program_id(0)

    @pl.when(t == 0)
    def _():
        acc_ref[...] = jnp.zeros_like(acc_ref)

    x = x_ref[0]
    bl = x.shape[-1]
    s1 = jnp.sum(x.reshape(C * nh * P, nw, P, bl), axis=1)
    s2 = jnp.sum(s1.reshape(C, nh, P, P, bl), axis=1)
    pp = s2.reshape(C * P * P, bl).astype(jnp.bfloat16)
    ft = jnp.dot(w_ref[...], pp, preferred_element_type=jnp.float32)
    ssum = jnp.sum(ft * ft, axis=0, keepdims=True)
    m = mask_ref[0]
    acc_ref[...] += ft * (jax.lax.rsqrt(ssum) * m)

    @pl.when(t == T - 1)
    def _():
        pooled = acc_ref[...]
        pinv = jax.lax.rsqrt(jnp.sum(pooled * pooled, axis=0, keepdims=True))
        vf = pooled * pinv

        tok = tok_ref[...]
        b, v = tok.shape[0], emb_ref.shape[0]
        viota = jax.lax.broadcasted_iota(jnp.int32, (b, v), 1)
        counts = jnp.zeros((b, v), jnp.float32)
        for l in range(L):
            counts += (viota == tok[:, l][:, None]).astype(jnp.float32)
        xt = jnp.dot((counts * (1.0 / L)).astype(jnp.bfloat16),
                     emb_ref[...].astype(jnp.bfloat16),
                     preferred_element_type=jnp.float32)
        xt += jnp.mean(pos_ref[0:L], axis=0, keepdims=True)
        seq = jnp.dot(xt.astype(jnp.bfloat16),
                      wt_ref[...].astype(jnp.bfloat16),
                      preferred_element_type=jnp.float32)
        tinv = jax.lax.rsqrt(jnp.sum(seq * seq, axis=-1, keepdims=True))
        tn = seq * tinv

        scale = jnp.exp(ls_ref[0, 0])
        sim = scale * jnp.dot(tn, vf,
                              preferred_element_type=jnp.float32)
        r = jax.lax.broadcasted_iota(jnp.int32, (b, b), 0)
        c = jax.lax.broadcasted_iota(jnp.int32, (b, b), 1)
        diag = jnp.sum(jnp.where(r == c, sim, 0.0))
        mr = jnp.max(sim, axis=1, keepdims=True)
        racc = jnp.sum(jnp.log(jnp.sum(jnp.exp(sim - mr), axis=1,
                                       keepdims=True)) + mr)
        mc = jnp.max(sim, axis=0, keepdims=True)
        cacc = jnp.sum(jnp.log(jnp.sum(jnp.exp(sim - mc), axis=0,
                                       keepdims=True)) + mc)
        loss = ((racc - diag) + (cacc - diag)) * (0.5 * inv_b)
        loss_ref[...] = jnp.reshape(loss, (1, 1))


def kernel(tok_emb, pos_emb, w_text, w_patch, logit_scale,
           text_input, video, video_mask):
    B, L = text_input.shape
    _, T, C, H, W = video.shape
    D = w_patch.shape[1]
    V, Kt = tok_emb.shape
    P = int(round((w_patch.shape[0] // C) ** 0.5))
    nh, nw = H // P, W // P
    CHW = C * H * W

    wp_t = ((w_patch.T) * (1.0 / (nh * nw))).astype(jnp.bfloat16)

    xs = video.transpose(1, 2, 3, 4, 0).reshape(T, CHW, B)
    mask_t = video_mask.astype(jnp.float32).T.reshape(T, 1, B)

    loss = pl.pallas_call(
        functools.partial(_clip_kernel, C=C, P=P, nh=nh, nw=nw, T=T, L=L,
                          inv_b=1.0 / B),
        out_shape=jax.ShapeDtypeStruct((1, 1), jnp.float32),
        grid_spec=pltpu.PrefetchScalarGridSpec(
            num_scalar_prefetch=0,
            grid=(T,),
            in_specs=[pl.BlockSpec((1, CHW, B), lambda t: (t, 0, 0)),
                      pl.BlockSpec((D, C * P * P), lambda t: (0, 0)),
                      pl.BlockSpec((1, 1, B), lambda t: (t, 0, 0)),
                      pl.BlockSpec((B, L), lambda t: (0, 0)),
                      pl.BlockSpec((V, Kt), lambda t: (0, 0)),
                      pl.BlockSpec(pos_emb.shape, lambda t: (0, 0)),
                      pl.BlockSpec((Kt, D), lambda t: (0, 0)),
                      pl.BlockSpec((1, 1), lambda t: (0, 0))],
            out_specs=pl.BlockSpec((1, 1), lambda t: (0, 0)),
            scratch_shapes=[pltpu.VMEM((D, B), jnp.float32)]),
        compiler_params=pltpu.CompilerParams(
            dimension_semantics=("arbitrary",),
            vmem_limit_bytes=64 * 1024 * 1024),
        cost_estimate=pl.CostEstimate(
            flops=T * CHW * B + 2 * T * C * P * P * B * D + 2 * B * B * D,
            transcendentals=2 * B * B,
            bytes_accessed=T * CHW * B * 4 + V * Kt * 4),
    )(xs, wp_t, mask_t, text_input, tok_emb, pos_emb, w_text,
      logit_scale.reshape(1, 1))
    return loss[0, 0]


# speedup vs baseline: 1.0753x; 1.0020x over previous
import functools

import jax
import jax.numpy as jnp
from jax.experimental import pallas as pl
from jax.experimental.pallas import tpu as pltpu


def _clip_kernel(x_ref, w_ref, mask_ref, tok_ref, emb_ref, pos_ref, wt_ref,
                 ls_ref, loss_ref, acc_ref, tn_ref,
                 *, C, P, nh, nw, T, L, inv_b):
    t = pl.program_id(0)

    @pl.when(t == 0)
    def _():
        acc_ref[...] = jnp.zeros_like(acc_ref)
        tok = tok_ref[...]
        b, v = tok.shape[0], emb_ref.shape[0]
        viota = jax.lax.broadcasted_iota(jnp.int32, (b, v), 1)
        counts = jnp.zeros((b, v), jnp.float32)
        for l in range(L):
            counts += (viota == tok[:, l][:, None]).astype(jnp.float32)
        xt = jnp.dot((counts * (1.0 / L)).astype(jnp.bfloat16),
                     emb_ref[...].astype(jnp.bfloat16),
                     preferred_element_type=jnp.float32)
        xt += jnp.mean(pos_ref[0:L], axis=0, keepdims=True)
        seq = jnp.dot(xt.astype(jnp.bfloat16),
                      wt_ref[...].astype(jnp.bfloat16),
                      preferred_element_type=jnp.float32)
        tinv = jax.lax.rsqrt(jnp.sum(seq * seq, axis=-1, keepdims=True))
        tn_ref[...] = seq * tinv

    x = x_ref[0]
    bl = x.shape[-1]
    s1 = jnp.sum(x.reshape(C * nh * P, nw, P, bl), axis=1)
    s2 = jnp.sum(s1.reshape(C, nh, P, P, bl), axis=1)
    pp = s2.reshape(C * P * P, bl).astype(jnp.bfloat16)
    ft = jnp.dot(w_ref[...], pp, preferred_element_type=jnp.float32)
    ssum = jnp.sum(ft * ft, axis=0, keepdims=True)
    m = mask_ref[0]
    acc_ref[...] += ft * (jax.lax.rsqrt(ssum) * m)

    @pl.when(t == T - 1)
    def _():
        pooled = acc_ref[...]
        pinv = jax.lax.rsqrt(jnp.sum(pooled * pooled, axis=0, keepdims=True))
        vf = pooled * pinv
        tn = tn_ref[...]
        b = tn.shape[0]
        scale = jnp.exp(ls_ref[0, 0])
        sim = scale * jnp.dot(tn, vf,
                              preferred_element_type=jnp.float32)
        r = jax.lax.broadcasted_iota(jnp.int32, (b, b), 0)
        c = jax.lax.broadcasted_iota(jnp.int32, (b, b), 1)
        diag = jnp.sum(jnp.where(r == c, sim, 0.0))
        mr = jnp.max(sim, axis=1, keepdims=True)
        racc = jnp.sum(jnp.log(jnp.sum(jnp.exp(sim - mr), axis=1,
                                       keepdims=True)) + mr)
        mc = jnp.max(sim, axis=0, keepdims=True)
        cacc = jnp.sum(jnp.log(jnp.sum(jnp.exp(sim - mc), axis=0,
                                       keepdims=True)) + mc)
        loss = ((racc - diag) + (cacc - diag)) * (0.5 * inv_b)
        loss_ref[...] = jnp.reshape(loss, (1, 1))


def kernel(tok_emb, pos_emb, w_text, w_patch, logit_scale,
           text_input, video, video_mask):
    B, L = text_input.shape
    _, T, C, H, W = video.shape
    D = w_patch.shape[1]
    V, Kt = tok_emb.shape
    P = int(round((w_patch.shape[0] // C) ** 0.5))
    nh, nw = H // P, W // P
    CHW = C * H * W

    wp_t = ((w_patch.T) * (1.0 / (nh * nw))).astype(jnp.bfloat16)

    xs = video.transpose(1, 2, 3, 4, 0).reshape(T, CHW, B)
    mask_t = video_mask.astype(jnp.float32).T.reshape(T, 1, B)

    loss = pl.pallas_call(
        functools.partial(_clip_kernel, C=C, P=P, nh=nh, nw=nw, T=T, L=L,
                          inv_b=1.0 / B),
        out_shape=jax.ShapeDtypeStruct((1, 1), jnp.float32),
        grid_spec=pltpu.PrefetchScalarGridSpec(
            num_scalar_prefetch=0,
            grid=(T,),
            in_specs=[pl.BlockSpec((1, CHW, B), lambda t: (t, 0, 0)),
                      pl.BlockSpec((D, C * P * P), lambda t: (0, 0)),
                      pl.BlockSpec((1, 1, B), lambda t: (t, 0, 0)),
                      pl.BlockSpec((B, L), lambda t: (0, 0)),
                      pl.BlockSpec((V, Kt), lambda t: (0, 0)),
                      pl.BlockSpec(pos_emb.shape, lambda t: (0, 0)),
                      pl.BlockSpec((Kt, D), lambda t: (0, 0)),
                      pl.BlockSpec((1, 1), lambda t: (0, 0))],
            out_specs=pl.BlockSpec((1, 1), lambda t: (0, 0)),
            scratch_shapes=[pltpu.VMEM((D, B), jnp.float32),
                            pltpu.VMEM((B, D), jnp.float32)]),
        compiler_params=pltpu.CompilerParams(
            dimension_semantics=("arbitrary",),
            vmem_limit_bytes=64 * 1024 * 1024),
        cost_estimate=pl.CostEstimate(
            flops=T * CHW * B + 2 * T * C * P * P * B * D + 2 * B * B * D,
            transcendentals=2 * B * B,
            bytes_accessed=T * CHW * B * 4 + V * Kt * 4),
    )(xs, wp_t, mask_t, text_input, tok_emb, pos_emb, w_text,
      logit_scale.reshape(1, 1))
    return loss[0, 0]


# speedup vs baseline: 1.0835x; 1.0076x over previous
import functools

import jax
import jax.numpy as jnp
from jax.experimental import pallas as pl
from jax.experimental.pallas import tpu as pltpu


def _clip_kernel(x_ref, w_ref, mask_ref, tok_ref, emb_ref, pos_ref, wt_ref,
                 ls_ref, loss_ref, acc_ref, tn_ref,
                 *, C, P, nh, nw, T, L, inv_b):
    t = pl.program_id(0)
    j = pl.program_id(1)
    nb = pl.num_programs(1)

    @pl.when((t == 0) & (j == 0))
    def _():
        acc_ref[...] = jnp.zeros_like(acc_ref)
        tok = tok_ref[...]
        b, v = tok.shape[0], emb_ref.shape[0]
        viota = jax.lax.broadcasted_iota(jnp.int32, (b, v), 1)
        counts = jnp.zeros((b, v), jnp.float32)
        for l in range(L):
            counts += (viota == tok[:, l][:, None]).astype(jnp.float32)
        xt = jnp.dot((counts * (1.0 / L)).astype(jnp.bfloat16),
                     emb_ref[...].astype(jnp.bfloat16),
                     preferred_element_type=jnp.float32)
        xt += jnp.mean(pos_ref[0:L], axis=0, keepdims=True)
        seq = jnp.dot(xt.astype(jnp.bfloat16),
                      wt_ref[...].astype(jnp.bfloat16),
                      preferred_element_type=jnp.float32)
        tinv = jax.lax.rsqrt(jnp.sum(seq * seq, axis=-1, keepdims=True))
        tn_ref[...] = seq * tinv

    x = x_ref[0]
    bl = x.shape[-1]
    s1 = jnp.sum(x.reshape(C * nh * P, nw, P, bl), axis=1)
    s2 = jnp.sum(s1.reshape(C, nh, P, P, bl), axis=1)
    pp = s2.reshape(C * P * P, bl).astype(jnp.bfloat16)
    ft = jnp.dot(w_ref[...], pp, preferred_element_type=jnp.float32)
    ssum = jnp.sum(ft * ft, axis=0, keepdims=True)
    m = mask_ref[0]
    acc_ref[:, pl.ds(j * bl, bl)] += ft * (jax.lax.rsqrt(ssum) * m)

    @pl.when((t == T - 1) & (j == nb - 1))
    def _():
        pooled = acc_ref[...]
        pinv = jax.lax.rsqrt(jnp.sum(pooled * pooled, axis=0, keepdims=True))
        vf = pooled * pinv
        tn = tn_ref[...]
        b = tn.shape[0]
        scale = jnp.exp(ls_ref[0, 0])
        sim = scale * jnp.dot(tn, vf,
                              preferred_element_type=jnp.float32)
        r = jax.lax.broadcasted_iota(jnp.int32, (b, b), 0)
        c = jax.lax.broadcasted_iota(jnp.int32, (b, b), 1)
        diag = jnp.sum(jnp.where(r == c, sim, 0.0))
        mr = jnp.max(sim, axis=1, keepdims=True)
        racc = jnp.sum(jnp.log(jnp.sum(jnp.exp(sim - mr), axis=1,
                                       keepdims=True)) + mr)
        mc = jnp.max(sim, axis=0, keepdims=True)
        cacc = jnp.sum(jnp.log(jnp.sum(jnp.exp(sim - mc), axis=0,
                                       keepdims=True)) + mc)
        loss = ((racc - diag) + (cacc - diag)) * (0.5 * inv_b)
        loss_ref[...] = jnp.reshape(loss, (1, 1))


def kernel(tok_emb, pos_emb, w_text, w_patch, logit_scale,
           text_input, video, video_mask):
    B, L = text_input.shape
    _, T, C, H, W = video.shape
    D = w_patch.shape[1]
    V, Kt = tok_emb.shape
    P = int(round((w_patch.shape[0] // C) ** 0.5))
    nh, nw = H // P, W // P
    CHW = C * H * W

    wp_t = ((w_patch.T) * (1.0 / (nh * nw))).astype(jnp.bfloat16)

    xs = video.transpose(1, 2, 3, 4, 0).reshape(T, CHW, B)
    mask_t = video_mask.astype(jnp.float32).T.reshape(T, 1, B)

    NB = 2 if B % 256 == 0 else 1
    bl = B // NB

    loss = pl.pallas_call(
        functools.partial(_clip_kernel, C=C, P=P, nh=nh, nw=nw, T=T, L=L,
                          inv_b=1.0 / B),
        out_shape=jax.ShapeDtypeStruct((1, 1), jnp.float32),
        grid_spec=pltpu.PrefetchScalarGridSpec(
            num_scalar_prefetch=0,
            grid=(T, NB),
            in_specs=[pl.BlockSpec((1, CHW, bl), lambda t, j: (t, 0, j)),
                      pl.BlockSpec((D, C * P * P), lambda t, j: (0, 0)),
                      pl.BlockSpec((1, 1, bl), lambda t, j: (t, 0, j)),
                      pl.BlockSpec((B, L), lambda t, j: (0, 0)),
                      pl.BlockSpec((V, Kt), lambda t, j: (0, 0)),
                      pl.BlockSpec(pos_emb.shape, lambda t, j: (0, 0)),
                      pl.BlockSpec((Kt, D), lambda t, j: (0, 0)),
                      pl.BlockSpec((1, 1), lambda t, j: (0, 0))],
            out_specs=pl.BlockSpec((1, 1), lambda t, j: (0, 0)),
            scratch_shapes=[pltpu.VMEM((D, B), jnp.float32),
                            pltpu.VMEM((B, D), jnp.float32)]),
        compiler_params=pltpu.CompilerParams(
            dimension_semantics=("arbitrary", "arbitrary"),
            vmem_limit_bytes=64 * 1024 * 1024),
        cost_estimate=pl.CostEstimate(
            flops=T * CHW * B + 2 * T * C * P * P * B * D + 2 * B * B * D,
            transcendentals=2 * B * B,
            bytes_accessed=T * CHW * B * 4 + V * Kt * 4),
    )(xs, wp_t, mask_t, text_input, tok_emb, pos_emb, w_text,
      logit_scale.reshape(1, 1))
    return loss[0, 0]


# speedup vs baseline: 1.1129x; 1.0271x over previous
import functools

import jax
import jax.numpy as jnp
from jax.experimental import pallas as pl
from jax.experimental.pallas import tpu as pltpu


def _clip_kernel(x_ref, w_ref, mask_ref, tok_ref, emb_ref, pos_ref, wt_ref,
                 ls_ref, loss_ref, acc_ref, tn_ref,
                 *, C, P, nh, nw, T, L, inv_b):
    t = pl.program_id(0)
    j = pl.program_id(1)
    nb = pl.num_programs(1)

    @pl.when((t == 0) & (j == 0))
    def _():
        acc_ref[...] = jnp.zeros_like(acc_ref)
        tok = tok_ref[...]
        b, v = tok.shape[0], emb_ref.shape[0]
        viota = jax.lax.broadcasted_iota(jnp.int32, (b, v), 1)
        counts = jnp.zeros((b, v), jnp.float32)
        for l in range(L):
            counts += (viota == tok[:, l][:, None]).astype(jnp.float32)
        xt = jnp.dot((counts * (1.0 / L)).astype(jnp.bfloat16),
                     emb_ref[...].astype(jnp.bfloat16),
                     preferred_element_type=jnp.float32)
        xt += jnp.mean(pos_ref[0:L], axis=0, keepdims=True)
        seq = jnp.dot(xt.astype(jnp.bfloat16),
                      wt_ref[...].astype(jnp.bfloat16),
                      preferred_element_type=jnp.float32)
        tinv = jax.lax.rsqrt(jnp.sum(seq * seq, axis=-1, keepdims=True))
        tn_ref[...] = seq * tinv

    x = x_ref[0]
    bl = x.shape[-1]
    s1 = jnp.sum(x.reshape(C * nh * P, nw, P, bl), axis=1)
    s2 = jnp.sum(s1.reshape(C, nh, P, P, bl), axis=1)
    pp = s2.reshape(C * P * P, bl).astype(jnp.bfloat16)
    ft = jnp.dot(w_ref[...], pp, preferred_element_type=jnp.float32)
    ssum = jnp.sum(ft * ft, axis=0, keepdims=True)
    m = mask_ref[t, :, pl.ds(j * bl, bl)]
    acc_ref[:, pl.ds(j * bl, bl)] += ft * (jax.lax.rsqrt(ssum) * m)

    @pl.when((t == T - 1) & (j == nb - 1))
    def _():
        pooled = acc_ref[...]
        pinv = jax.lax.rsqrt(jnp.sum(pooled * pooled, axis=0, keepdims=True))
        vf = pooled * pinv
        tn = tn_ref[...]
        b = tn.shape[0]
        scale = jnp.exp(ls_ref[0, 0])
        sim = scale * jnp.dot(tn, vf,
                              preferred_element_type=jnp.float32)
        r = jax.lax.broadcasted_iota(jnp.int32, (b, b), 0)
        c = jax.lax.broadcasted_iota(jnp.int32, (b, b), 1)
        diag = jnp.sum(jnp.where(r == c, sim, 0.0))
        mr = jnp.max(sim, axis=1, keepdims=True)
        racc = jnp.sum(jnp.log(jnp.sum(jnp.exp(sim - mr), axis=1,
                                       keepdims=True)) + mr)
        mc = jnp.max(sim, axis=0, keepdims=True)
        cacc = jnp.sum(jnp.log(jnp.sum(jnp.exp(sim - mc), axis=0,
                                       keepdims=True)) + mc)
        loss = ((racc - diag) + (cacc - diag)) * (0.5 * inv_b)
        loss_ref[...] = jnp.reshape(loss, (1, 1))


def kernel(tok_emb, pos_emb, w_text, w_patch, logit_scale,
           text_input, video, video_mask):
    B, L = text_input.shape
    _, T, C, H, W = video.shape
    D = w_patch.shape[1]
    V, Kt = tok_emb.shape
    P = int(round((w_patch.shape[0] // C) ** 0.5))
    nh, nw = H // P, W // P
    CHW = C * H * W

    wp_t = ((w_patch.T) * (1.0 / (nh * nw))).astype(jnp.bfloat16)

    xs = video.transpose(1, 2, 3, 4, 0).reshape(T, CHW, B)
    mask_t = video_mask.astype(jnp.float32).T.reshape(T, 1, B)

    NB = 2 if B % 256 == 0 else 1
    bl = B // NB

    loss = pl.pallas_call(
        functools.partial(_clip_kernel, C=C, P=P, nh=nh, nw=nw, T=T, L=L,
                          inv_b=1.0 / B),
        out_shape=jax.ShapeDtypeStruct((1, 1), jnp.float32),
        grid_spec=pltpu.PrefetchScalarGridSpec(
            num_scalar_prefetch=0,
            grid=(T, NB),
            in_specs=[pl.BlockSpec((1, CHW, bl), lambda t, j: (t, 0, j)),
                      pl.BlockSpec((D, C * P * P), lambda t, j: (0, 0)),
                      pl.BlockSpec((T, 1, B), lambda t, j: (0, 0, 0)),
                      pl.BlockSpec((B, L), lambda t, j: (0, 0)),
                      pl.BlockSpec((V, Kt), lambda t, j: (0, 0)),
                      pl.BlockSpec(pos_emb.shape, lambda t, j: (0, 0)),
                      pl.BlockSpec((Kt, D), lambda t, j: (0, 0)),
                      pl.BlockSpec((1, 1), lambda t, j: (0, 0))],
            out_specs=pl.BlockSpec((1, 1), lambda t, j: (0, 0)),
            scratch_shapes=[pltpu.VMEM((D, B), jnp.float32),
                            pltpu.VMEM((B, D), jnp.float32)]),
        compiler_params=pltpu.CompilerParams(
            dimension_semantics=("arbitrary", "arbitrary"),
            vmem_limit_bytes=64 * 1024 * 1024),
        cost_estimate=pl.CostEstimate(
            flops=T * CHW * B + 2 * T * C * P * P * B * D + 2 * B * B * D,
            transcendentals=2 * B * B,
            bytes_accessed=T * CHW * B * 4 + V * Kt * 4),
    )(xs, wp_t, mask_t, text_input, tok_emb, pos_emb, w_text,
      logit_scale.reshape(1, 1))
    return loss[0, 0]
